# baseline (speedup 1.0000x reference)
import jax
import jax.numpy as jnp
import numpy as np
from jax import lax
from jax.experimental import pallas as pl
from jax.experimental.pallas import tpu as pltpu
from jax.experimental.pallas import tpu_sc as plsc

NUM_CLASSES = 1000
SMOOTHING = 0.1
FILL = float(np.float32(SMOOTHING / NUM_CLASSES))
PEAK = float(np.float32(np.float32(SMOOTHING / NUM_CLASSES) + np.float32(1.0 - SMOOTHING)))

NC, NS, L = 2, 16, 16
NW = NC * NS
BATCH = 16384
RPW = BATCH // NW
R = 32
NCHUNK = RPW // R
CW = R * NUM_CLASSES


def _sc_body(tgt_hbm, out_hbm, tgt_v, buf0, buf1, sem0, sem1):
    wid = lax.axis_index("s") * NC + lax.axis_index("c")
    base_row = wid * RPW
    pltpu.sync_copy(tgt_hbm.at[pl.ds(base_row, RPW)], tgt_v)

    fill_vec = jnp.full((L,), FILL, jnp.float32)
    peak_vec = jnp.full((L,), PEAK, jnp.float32)
    lanes = lax.iota(jnp.int32, L)

    def fill_buf(buf):
        def body(i, carry):
            buf[pl.ds(i * L, L)] = fill_vec
            return carry
        lax.fori_loop(0, CW // L, body, 0)

    fill_buf(buf0)
    fill_buf(buf1)

    bufs = (buf0, buf1)
    sems = (sem0, sem1)

    def positions(b):
        out = []
        for k in range(R // L):
            t = tgt_v[pl.ds(b * R + k * L, L)]
            pos = (k * L + lanes) * NUM_CLASSES + t
            out.append(pos)
        return out

    copies = [None, None]
    for b in range(NCHUNK):
        p = b % 2
        if copies[p] is not None:
            copies[p].wait()
            for pos in positions(b - 2):
                plsc.store_scatter(bufs[p], [pos], fill_vec)
        for pos in positions(b):
            plsc.store_scatter(bufs[p], [pos], peak_vec)
        dst = out_hbm.at[pl.ds((base_row + b * R) * NUM_CLASSES, CW)]
        copies[p] = pltpu.async_copy(bufs[p], dst, sems[p])
    copies[0].wait()
    copies[1].wait()


def kernel(target, pred):
    batch = target.shape[0]
    mesh = plsc.VectorSubcoreMesh(core_axis_name="c", subcore_axis_name="s")
    out_flat = pl.kernel(
        _sc_body,
        out_type=jax.ShapeDtypeStruct((batch * NUM_CLASSES,), jnp.float32),
        mesh=mesh,
        compiler_params=pltpu.CompilerParams(needs_layout_passes=False),
        scratch_types=[
            pltpu.VMEM((RPW,), jnp.int32),
            pltpu.VMEM((CW,), jnp.float32),
            pltpu.VMEM((CW,), jnp.float32),
            pltpu.SemaphoreType.DMA,
            pltpu.SemaphoreType.DMA,
        ],
    )(target)
    return out_flat.reshape(batch, NUM_CLASSES)


# speedup vs baseline: 1.6154x; 1.6154x over previous
import jax
import jax.numpy as jnp
import numpy as np
from jax import lax
from jax.experimental import pallas as pl
from jax.experimental.pallas import tpu as pltpu
from jax.experimental.pallas import tpu_sc as plsc

NUM_CLASSES = 1000
SMOOTHING = 0.1
FILL = float(np.float32(SMOOTHING / NUM_CLASSES))
PEAK = float(np.float32(np.float32(SMOOTHING / NUM_CLASSES) + np.float32(1.0 - SMOOTHING)))

NC, NS, L = 2, 16, 16
NW = NC * NS
BATCH = 16384
RPW = BATCH // NW
R = 32
NCHUNK = RPW // R


def _sc_body(tgt_hbm, fill_hbm, out_hbm, tgt_v, buf0, buf1, sem0, sem1, semf):
    wid = lax.axis_index("s") * NC + lax.axis_index("c")
    base_row = wid * RPW

    fa = pltpu.async_copy(fill_hbm, buf0, semf)
    fb = pltpu.async_copy(fill_hbm, buf1, semf)
    pltpu.sync_copy(tgt_hbm.at[pl.ds(base_row, RPW)], tgt_v)
    fa.wait()
    fb.wait()

    fill_vec = jnp.full((L,), FILL, jnp.float32)
    peak_vec = jnp.full((L,), PEAK, jnp.float32)
    lanes = lax.iota(jnp.int32, L)

    bufs = (buf0, buf1)
    sems = (sem0, sem1)

    def indices(b):
        out = []
        for k in range(R // L):
            t = tgt_v[pl.ds(b * R + k * L, L)]
            out.append((k * L + lanes, t))
        return out

    copies = [None, None]
    for b in range(NCHUNK):
        p = b % 2
        if copies[p] is not None:
            copies[p].wait()
            for rows, cols in indices(b - 2):
                plsc.store_scatter(bufs[p], [rows, cols], fill_vec)
        for rows, cols in indices(b):
            plsc.store_scatter(bufs[p], [rows, cols], peak_vec)
        dst = out_hbm.at[pl.ds(base_row + b * R, R), :]
        copies[p] = pltpu.async_copy(bufs[p], dst, sems[p])
    copies[0].wait()
    copies[1].wait()


def kernel(target, pred):
    batch = target.shape[0]
    fill_const = jnp.full((R, NUM_CLASSES), FILL, jnp.float32)
    mesh = plsc.VectorSubcoreMesh(core_axis_name="c", subcore_axis_name="s")
    return pl.kernel(
        _sc_body,
        out_type=jax.ShapeDtypeStruct((batch, NUM_CLASSES), jnp.float32),
        mesh=mesh,
        compiler_params=pltpu.CompilerParams(needs_layout_passes=False),
        scratch_types=[
            pltpu.VMEM((RPW,), jnp.int32),
            pltpu.VMEM((R, NUM_CLASSES), jnp.float32),
            pltpu.VMEM((R, NUM_CLASSES), jnp.float32),
            pltpu.SemaphoreType.DMA,
            pltpu.SemaphoreType.DMA,
            pltpu.SemaphoreType.DMA,
        ],
    )(target, fill_const)
